# Initial kernel scaffold; baseline (speedup 1.0000x reference)
import jax
import jax.numpy as jnp
from jax.experimental import pallas as pl


def kernel(labels_tensor, coords_tensor):
    raise NotImplementedError("write your pallas kernel here")


# speedup vs baseline: 6.5667x; 6.5667x over previous
import jax
import jax.numpy as jnp
import numpy as np
from jax.experimental import pallas as pl

RCR = 5.2
RCA = 3.5
ETA_R = 16.0
ETA_A = 8.0
N = 48
NS = 4
NP = 10
NZ = 8
NA = 4
NAZ = NA * NZ
NI = 8
JK = N * N


def _pow_zeta(t):
    t = t * t
    t = t * t
    t = t * t
    t = t * t
    return t * t


def _aev_kernel(lab_ref, sjT_ref, skT_ref, cjf_ref, ckf_ref, ct_ref, c48_ref,
                jidx_ref, kidx_ref, rad_ref, ang_ref):
    labf = lab_ref[0].astype(jnp.float32)
    sjT = sjT_ref[0]
    skT = skT_ref[0]
    cjf = cjf_ref[0]
    ckf = ckf_ref[0]
    ct = ct_ref[0]
    c48 = c48_ref[0]
    jidx = jidx_ref[0]
    kidx = kidx_ref[0]

    sp_iota = jax.lax.broadcasted_iota(jnp.int32, (NS, N), 0).astype(jnp.float32)
    oh4 = (labf == sp_iota).astype(jnp.float32)

    smin = jnp.minimum(sjT, skT)
    smax = jnp.maximum(sjT, skT)
    clsT = smin * (7.0 - smin) * 0.5 + smax
    p_iota = jax.lax.broadcasted_iota(jnp.int32, (JK, NP), 1).astype(jnp.float32)
    oh10T = (clsT == p_iota).astype(jnp.float32)

    neq = (jidx != kidx).astype(jnp.float32)

    shf_z = (np.pi / 16.0) + (np.pi / 8.0) * \
        jax.lax.broadcasted_iota(jnp.int32, (1, NZ, 1), 1).astype(jnp.float32)
    cos_sz = jnp.cos(shf_z)
    sin_sz = jnp.sin(shf_z)
    shf_a = 0.9 + 0.65 * \
        jax.lax.broadcasted_iota(jnp.int32, (1, NA, 1), 1).astype(jnp.float32)
    shf_r = 0.9 + 0.26875 * \
        jax.lax.broadcasted_iota(jnp.int32, (1, 16, 1), 1).astype(jnp.float32)

    cx = ct[0:1, :]
    cy = ct[1:2, :]
    cz = ct[2:3, :]

    for i0 in range(0, N, NI):
        ci = c48[i0:i0 + NI, :]
        cix = ci[:, 0:1]
        ciy = ci[:, 1:2]
        ciz = ci[:, 2:3]
        i_row = (i0 + jax.lax.broadcasted_iota(jnp.int32, (NI, 1), 0)
                 ).astype(jnp.float32)

        dx = cx - cix
        dy = cy - ciy
        dz = cz - ciz
        d = jnp.sqrt(dx * dx + dy * dy + dz * dz + 1e-12)
        jmask = (jax.lax.broadcasted_iota(jnp.int32, (NI, N), 1)
                 != (i0 + jax.lax.broadcasted_iota(jnp.int32, (NI, N), 0))
                 ).astype(jnp.float32)
        fcr = jnp.where(d <= RCR, 0.5 * jnp.cos(jnp.pi / RCR * d) + 0.5,
                        0.0) * jmask
        rad3 = 0.25 * jnp.exp(-ETA_R * (d[:, None, :] - shf_r) ** 2) \
            * fcr[:, None, :]
        rparts = [jnp.sum(rad3 * oh4[s:s + 1][None], axis=2)
                  for s in range(NS)]
        rad_ref[0, i0:i0 + NI, :] = jnp.concatenate(rparts, axis=1)

        vjx = cjf[0:1, :] - cix
        vjy = cjf[1:2, :] - ciy
        vjz = cjf[2:3, :] - ciz
        vkx = ckf[0:1, :] - cix
        vky = ckf[1:2, :] - ciy
        vkz = ckf[2:3, :] - ciz
        dj = jnp.sqrt(vjx * vjx + vjy * vjy + vjz * vjz + 1e-12)
        dk = jnp.sqrt(vkx * vkx + vky * vky + vkz * vkz + 1e-12)
        dot = vjx * vkx + vjy * vky + vjz * vkz

        cosang = jnp.clip(0.95 * dot / jnp.maximum(dj * dk, 1e-10),
                          -1.0, 1.0)
        sinang = jnp.sqrt(jnp.maximum(1.0 - cosang * cosang, 0.0))
        davg = (dj + dk) * 0.5

        fcaj = jnp.where(dj <= RCA, 0.5 * jnp.cos(jnp.pi / RCA * dj) + 0.5,
                         0.0) * (jidx != i_row).astype(jnp.float32)
        fcak = jnp.where(dk <= RCA, 0.5 * jnp.cos(jnp.pi / RCA * dk) + 0.5,
                         0.0) * (kidx != i_row).astype(jnp.float32)
        w = fcaj * fcak * neq

        base = 0.5 + 0.5 * (cosang[:, None, :] * cos_sz +
                            sinang[:, None, :] * sin_sz)
        f1 = _pow_zeta(base)
        f2 = jnp.exp(-ETA_A * (davg[:, None, :] - shf_a) ** 2)

        term = (w[:, None, None, :] * f2[:, :, None, :] *
                f1[:, None, :, :]).reshape(NI * NAZ, JK)

        ang = jax.lax.dot_general(term, oh10T, (((1,), (0,)), ((), ())),
                                  preferred_element_type=jnp.float32)
        ang_ref[0, i0 * NAZ:(i0 + NI) * NAZ, :] = ang


def kernel(labels_tensor, coords_tensor):
    B = labels_tensor.shape[0]
    labs3 = labels_tensor.reshape(B, 1, N)
    labf = labels_tensor.astype(jnp.float32)
    sjT = jnp.broadcast_to(labf[:, :, None], (B, N, N)).reshape(B, JK, 1)
    skT = jnp.broadcast_to(labf[:, None, :], (B, N, N)).reshape(B, JK, 1)
    cjf = jnp.broadcast_to(coords_tensor[:, :, None, :], (B, N, N, 3)) \
        .reshape(B, JK, 3).transpose(0, 2, 1)
    ckf = jnp.broadcast_to(coords_tensor[:, None, :, :], (B, N, N, 3)) \
        .reshape(B, JK, 3).transpose(0, 2, 1)
    ct = coords_tensor.transpose(0, 2, 1)
    idx = jnp.arange(JK, dtype=jnp.int32)
    jidx = (idx // N).astype(jnp.float32).reshape(1, 1, JK)
    kidx = (idx % N).astype(jnp.float32).reshape(1, 1, JK)

    rad, ang = pl.pallas_call(
        _aev_kernel,
        grid=(B,),
        in_specs=[
            pl.BlockSpec((1, 1, N), lambda b: (b, 0, 0)),
            pl.BlockSpec((1, JK, 1), lambda b: (b, 0, 0)),
            pl.BlockSpec((1, JK, 1), lambda b: (b, 0, 0)),
            pl.BlockSpec((1, 3, JK), lambda b: (b, 0, 0)),
            pl.BlockSpec((1, 3, JK), lambda b: (b, 0, 0)),
            pl.BlockSpec((1, 3, N), lambda b: (b, 0, 0)),
            pl.BlockSpec((1, N, 3), lambda b: (b, 0, 0)),
            pl.BlockSpec((1, 1, JK), lambda b: (0, 0, 0)),
            pl.BlockSpec((1, 1, JK), lambda b: (0, 0, 0)),
        ],
        out_specs=[
            pl.BlockSpec((1, N, 64), lambda b: (b, 0, 0)),
            pl.BlockSpec((1, N * NAZ, NP), lambda b: (b, 0, 0)),
        ],
        out_shape=[
            jax.ShapeDtypeStruct((B, N, 64), jnp.float32),
            jax.ShapeDtypeStruct((B, N * NAZ, NP), jnp.float32),
        ],
    )(labs3, sjT, skT, cjf, ckf, ct, coords_tensor, jidx, kidx)

    angr = ang.reshape(B, N, NAZ, NP).transpose(0, 1, 3, 2).reshape(B, N, NP * NAZ)
    aev = jnp.concatenate([rad, angr], axis=-1)
    return labels_tensor, aev


# speedup vs baseline: 11.1975x; 1.7052x over previous
import jax
import jax.numpy as jnp
import numpy as np
from jax.experimental import pallas as pl

RCR = 5.2
RCA = 3.5
ETA_R = 16.0
ETA_A = 8.0
N = 48
NS = 4
NP = 10
NZ = 8
NA = 4
NAZ = NA * NZ
NI = 8
NPAIR = N * (N - 1) // 2
JK = 1152

_JL, _KL = np.triu_indices(N, k=1)
_JL = np.concatenate([_JL, np.zeros(JK - NPAIR, np.int64)]).astype(np.int32)
_KL = np.concatenate([_KL, np.zeros(JK - NPAIR, np.int64)]).astype(np.int32)

_C1 = float(np.exp(-8.0 * 0.65 ** 2))
_C2 = float(np.exp(-8.0 * (2 * 0.65) ** 2))
_C3 = float(np.exp(-8.0 * (3 * 0.65) ** 2))


def _pow_zeta(t):
    t = t * t
    t = t * t
    t = t * t
    t = t * t
    return t * t


def _aev_kernel(lab_ref, sjT_ref, skT_ref, cjf_ref, ckf_ref, ct_ref, c48_ref,
                jidx_ref, kidx_ref, rad_ref, ang_ref):
    labf = lab_ref[0].astype(jnp.float32)
    sjT = sjT_ref[0]
    skT = skT_ref[0]
    cjf = cjf_ref[0]
    ckf = ckf_ref[0]
    ct = ct_ref[0]
    c48 = c48_ref[0]
    jidx = jidx_ref[0]
    kidx = kidx_ref[0]

    sp_iota = jax.lax.broadcasted_iota(jnp.int32, (NS, N), 0).astype(jnp.float32)
    oh4 = (labf == sp_iota).astype(jnp.float32)

    smin = jnp.minimum(sjT, skT)
    smax = jnp.maximum(sjT, skT)
    clsT = smin * (7.0 - smin) * 0.5 + smax
    p_iota = jax.lax.broadcasted_iota(jnp.int32, (JK, NP), 1).astype(jnp.float32)
    oh10T = (clsT == p_iota).astype(jnp.float32)

    pairw = jnp.where(jidx != kidx, 2.0, 0.0)

    shf_z = (np.pi / 16.0) + (np.pi / 8.0) * \
        jax.lax.broadcasted_iota(jnp.int32, (1, NZ, 1), 1).astype(jnp.float32)
    cos_sz = jnp.cos(shf_z)
    sin_sz = jnp.sin(shf_z)
    shf_r = 0.9 + 0.26875 * \
        jax.lax.broadcasted_iota(jnp.int32, (1, 16, 1), 1).astype(jnp.float32)

    cx = ct[0:1, :]
    cy = ct[1:2, :]
    cz = ct[2:3, :]

    for i0 in range(0, N, NI):
        ci = c48[i0:i0 + NI, :]
        cix = ci[:, 0:1]
        ciy = ci[:, 1:2]
        ciz = ci[:, 2:3]
        i_row = (i0 + jax.lax.broadcasted_iota(jnp.int32, (NI, 1), 0)
                 ).astype(jnp.float32)

        dx = cx - cix
        dy = cy - ciy
        dz = cz - ciz
        d = jnp.sqrt(dx * dx + dy * dy + dz * dz + 1e-12)
        jmask = (jax.lax.broadcasted_iota(jnp.int32, (NI, N), 1)
                 != (i0 + jax.lax.broadcasted_iota(jnp.int32, (NI, N), 0))
                 ).astype(jnp.float32)
        fcr = jnp.where(d <= RCR, 0.5 * jnp.cos(jnp.pi / RCR * d) + 0.5,
                        0.0) * jmask
        rad3 = 0.25 * jnp.exp(-ETA_R * (d[:, None, :] - shf_r) ** 2) \
            * fcr[:, None, :]
        rparts = [jnp.sum(rad3 * oh4[s:s + 1][None], axis=2)
                  for s in range(NS)]
        rad_ref[0, i0:i0 + NI, :] = jnp.concatenate(rparts, axis=1)

        vjx = cjf[0:1, :] - cix
        vjy = cjf[1:2, :] - ciy
        vjz = cjf[2:3, :] - ciz
        vkx = ckf[0:1, :] - cix
        vky = ckf[1:2, :] - ciy
        vkz = ckf[2:3, :] - ciz
        dj = jnp.sqrt(vjx * vjx + vjy * vjy + vjz * vjz + 1e-12)
        dk = jnp.sqrt(vkx * vkx + vky * vky + vkz * vkz + 1e-12)
        dot = vjx * vkx + vjy * vky + vjz * vkz

        cosang = jnp.clip(0.95 * dot / jnp.maximum(dj * dk, 1e-10),
                          -1.0, 1.0)
        sinang = jnp.sqrt(jnp.maximum(1.0 - cosang * cosang, 0.0))

        fcaj = jnp.where(dj <= RCA, 0.5 * jnp.cos(jnp.pi / RCA * dj) + 0.5,
                         0.0) * (jidx != i_row).astype(jnp.float32)
        fcak = jnp.where(dk <= RCA, 0.5 * jnp.cos(jnp.pi / RCA * dk) + 0.5,
                         0.0) * (kidx != i_row).astype(jnp.float32)
        w = fcaj * fcak * pairw

        base = 0.5 + 0.5 * (cosang[:, None, :] * cos_sz +
                            sinang[:, None, :] * sin_sz)
        f1 = _pow_zeta(base)

        x = jnp.minimum((dj + dk) * 0.5, RCA) - 0.9
        f20 = jnp.exp(-ETA_A * x * x) * w
        e1 = jnp.exp((2.0 * ETA_A * 0.65) * x)
        g1 = f20 * e1
        g2 = g1 * e1
        g3 = g2 * e1
        wf2 = jnp.stack([f20, _C1 * g1, _C2 * g2, _C3 * g3], axis=1)

        term = (wf2[:, :, None, :] * f1[:, None, :, :]).reshape(NI * NAZ, JK)

        ang = jax.lax.dot_general(term, oh10T, (((1,), (0,)), ((), ())),
                                  preferred_element_type=jnp.float32)
        ang_ref[0, i0 * NAZ:(i0 + NI) * NAZ, :] = ang


def kernel(labels_tensor, coords_tensor):
    B = labels_tensor.shape[0]
    labs3 = labels_tensor.reshape(B, 1, N)
    labf = labels_tensor.astype(jnp.float32)
    jl = jnp.asarray(_JL)
    kl = jnp.asarray(_KL)
    sjT = labf[:, jl].reshape(B, JK, 1)
    skT = labf[:, kl].reshape(B, JK, 1)
    cjf = coords_tensor[:, jl, :].transpose(0, 2, 1)
    ckf = coords_tensor[:, kl, :].transpose(0, 2, 1)
    ct = coords_tensor.transpose(0, 2, 1)
    jidx = jl.astype(jnp.float32).reshape(1, 1, JK)
    kidx = kl.astype(jnp.float32).reshape(1, 1, JK)

    rad, ang = pl.pallas_call(
        _aev_kernel,
        grid=(B,),
        in_specs=[
            pl.BlockSpec((1, 1, N), lambda b: (b, 0, 0)),
            pl.BlockSpec((1, JK, 1), lambda b: (b, 0, 0)),
            pl.BlockSpec((1, JK, 1), lambda b: (b, 0, 0)),
            pl.BlockSpec((1, 3, JK), lambda b: (b, 0, 0)),
            pl.BlockSpec((1, 3, JK), lambda b: (b, 0, 0)),
            pl.BlockSpec((1, 3, N), lambda b: (b, 0, 0)),
            pl.BlockSpec((1, N, 3), lambda b: (b, 0, 0)),
            pl.BlockSpec((1, 1, JK), lambda b: (0, 0, 0)),
            pl.BlockSpec((1, 1, JK), lambda b: (0, 0, 0)),
        ],
        out_specs=[
            pl.BlockSpec((1, N, 64), lambda b: (b, 0, 0)),
            pl.BlockSpec((1, N * NAZ, NP), lambda b: (b, 0, 0)),
        ],
        out_shape=[
            jax.ShapeDtypeStruct((B, N, 64), jnp.float32),
            jax.ShapeDtypeStruct((B, N * NAZ, NP), jnp.float32),
        ],
    )(labs3, sjT, skT, cjf, ckf, ct, coords_tensor, jidx, kidx)

    angr = ang.reshape(B, N, NAZ, NP).transpose(0, 1, 3, 2).reshape(B, N, NP * NAZ)
    aev = jnp.concatenate([rad, angr], axis=-1)
    return labels_tensor, aev
